# baseline (speedup 1.0000x reference)
import jax
import jax.numpy as jnp
from jax.experimental import pallas as pl
from jax.experimental.pallas import tpu as pltpu

B, N, NFEAT, NHID, NCLASS = 4, 4096, 512, 512, 10
LANES = 128
RT = 512
NT = N // RT


def _s1_kernel(x_ref, w1_ref, s1_ref):
    s1_ref[0] = jnp.dot(x_ref[0].astype(jnp.bfloat16), w1_ref[...],
                        preferred_element_type=jnp.float32).astype(jnp.bfloat16)


def _gcn_kernel(s1_ref, adj_ref, b1_ref, w2_ref, wdt_ref,
                p_ref, s2_ref, gt_ref):
    t = pl.program_id(1)

    @pl.when(t == 0)
    def _init():
        gt_ref[...] = jnp.zeros_like(gt_ref)

    adj_t = adj_ref[0].astype(jnp.bfloat16)

    z = jnp.dot(adj_t, s1_ref[0], preferred_element_type=jnp.float32)
    h1_t = jnp.maximum(z + b1_ref[...], 0.0).astype(jnp.bfloat16)

    s2_t = jnp.dot(h1_t, w2_ref[...], preferred_element_type=jnp.float32)
    s2_ref[pl.ds(t * RT, RT), :] = s2_t.astype(jnp.bfloat16)

    wdt_t = wdt_ref[:, pl.ds(t * RT, RT)]
    gt_ref[...] += jnp.dot(wdt_t, adj_t,
                           preferred_element_type=jnp.float32)

    @pl.when(t == NT - 1)
    def _fin():
        p_ref[0] = jnp.dot(gt_ref[...].astype(jnp.bfloat16), s2_ref[...],
                           preferred_element_type=jnp.float32)


def kernel(x, adj, W1, b1, W2, b2, Wd, bd):
    w1 = W1.astype(jnp.bfloat16)
    w2 = jnp.pad(W2, ((0, 0), (0, LANES - NCLASS))).astype(jnp.bfloat16)
    wdt = jnp.pad(Wd.reshape(N, NCLASS * NCLASS),
                  ((0, 0), (0, LANES - NCLASS * NCLASS))).T.astype(jnp.bfloat16)
    b1r = b1.reshape(1, NHID)

    s1 = pl.pallas_call(
        _s1_kernel,
        grid=(B,),
        in_specs=[
            pl.BlockSpec((1, N, NFEAT), lambda b: (b, 0, 0)),
            pl.BlockSpec((NFEAT, NHID), lambda b: (0, 0)),
        ],
        out_specs=pl.BlockSpec((1, N, NHID), lambda b: (b, 0, 0)),
        out_shape=jax.ShapeDtypeStruct((B, N, NHID), jnp.bfloat16),
        compiler_params=pltpu.CompilerParams(
            dimension_semantics=("arbitrary",),
        ),
    )(x, w1)

    p = pl.pallas_call(
        _gcn_kernel,
        grid=(B, NT),
        in_specs=[
            pl.BlockSpec((1, N, NHID), lambda b, t: (b, 0, 0)),
            pl.BlockSpec((1, RT, N), lambda b, t: (b, t, 0)),
            pl.BlockSpec((1, NHID), lambda b, t: (0, 0)),
            pl.BlockSpec((NHID, LANES), lambda b, t: (0, 0)),
            pl.BlockSpec((LANES, N), lambda b, t: (0, 0)),
        ],
        out_specs=pl.BlockSpec((1, LANES, LANES), lambda b, t: (b, 0, 0)),
        out_shape=jax.ShapeDtypeStruct((B, LANES, LANES), jnp.float32),
        scratch_shapes=[
            pltpu.VMEM((N, LANES), jnp.bfloat16),
            pltpu.VMEM((LANES, N), jnp.float32),
        ],
        compiler_params=pltpu.CompilerParams(
            dimension_semantics=("arbitrary", "arbitrary"),
        ),
    )(s1, adj, b1r, w2, wdt)

    pr = p[:, :NCLASS * NCLASS, :NCLASS].reshape(B, NCLASS, NCLASS, NCLASS)
    out = jnp.einsum('bkck->bc', pr)
    wd3 = Wd.reshape(N, NCLASS, NCLASS)
    out = out + jnp.einsum('k,nkc->c', b2, wd3)[None, :] + bd[None, :]
    return out


# speedup vs baseline: 1.2841x; 1.2841x over previous
import jax
import jax.numpy as jnp
from jax.experimental import pallas as pl
from jax.experimental.pallas import tpu as pltpu

B, N, NFEAT, NHID, NCLASS = 4, 4096, 512, 512, 10
LANES = 128
RT = 512
NT = N // RT


def _gcn_kernel(x_ref, adj_ref, w1_ref, b1_ref, w2_ref, wdt_ref,
                p_ref, s1_ref, s2_ref, gt_ref):
    t = pl.program_id(1)

    @pl.when(t == 0)
    def _init():
        s1 = jnp.dot(x_ref[0].astype(jnp.bfloat16), w1_ref[...],
                     preferred_element_type=jnp.float32)
        s1_ref[...] = s1.astype(jnp.bfloat16)
        gt_ref[...] = jnp.zeros_like(gt_ref)

    adj_t = adj_ref[0].astype(jnp.bfloat16)

    z = jnp.dot(adj_t, s1_ref[...], preferred_element_type=jnp.float32)
    h1_t = jnp.maximum(z + b1_ref[...], 0.0).astype(jnp.bfloat16)

    s2_t = jnp.dot(h1_t, w2_ref[...], preferred_element_type=jnp.float32)
    s2_ref[pl.ds(t * RT, RT), :] = s2_t.astype(jnp.bfloat16)

    wdt_t = wdt_ref[:, pl.ds(t * RT, RT)]
    gt_ref[...] += jnp.dot(wdt_t, adj_t,
                           preferred_element_type=jnp.float32)

    @pl.when(t == NT - 1)
    def _fin():
        p_ref[0] = jnp.dot(gt_ref[...].astype(jnp.bfloat16), s2_ref[...],
                           preferred_element_type=jnp.float32)


def kernel(x, adj, W1, b1, W2, b2, Wd, bd):
    w1 = W1.astype(jnp.bfloat16)
    w2 = jnp.pad(W2, ((0, 0), (0, LANES - NCLASS))).astype(jnp.bfloat16)
    wdt = jnp.pad(Wd.reshape(N, NCLASS * NCLASS),
                  ((0, 0), (0, LANES - NCLASS * NCLASS))).T.astype(jnp.bfloat16)
    b1r = b1.reshape(1, NHID)

    p = pl.pallas_call(
        _gcn_kernel,
        grid=(B, NT),
        in_specs=[
            pl.BlockSpec((1, N, NFEAT), lambda b, t: (b, 0, 0)),
            pl.BlockSpec((1, RT, N), lambda b, t: (b, t, 0)),
            pl.BlockSpec((NFEAT, NHID), lambda b, t: (0, 0)),
            pl.BlockSpec((1, NHID), lambda b, t: (0, 0)),
            pl.BlockSpec((NHID, LANES), lambda b, t: (0, 0)),
            pl.BlockSpec((LANES, N), lambda b, t: (0, 0)),
        ],
        out_specs=pl.BlockSpec((1, LANES, LANES), lambda b, t: (b, 0, 0)),
        out_shape=jax.ShapeDtypeStruct((B, LANES, LANES), jnp.float32),
        scratch_shapes=[
            pltpu.VMEM((N, NHID), jnp.bfloat16),
            pltpu.VMEM((N, LANES), jnp.bfloat16),
            pltpu.VMEM((LANES, N), jnp.float32),
        ],
        compiler_params=pltpu.CompilerParams(
            dimension_semantics=("parallel", "arbitrary"),
        ),
    )(x, adj, w1, b1r, w2, wdt)

    pr = p[:, :NCLASS * NCLASS, :NCLASS].reshape(B, NCLASS, NCLASS, NCLASS)
    out = jnp.einsum('bkck->bc', pr)
    rs = jnp.sum(wdt.astype(jnp.float32), axis=1)[:NCLASS * NCLASS]
    out = out + (b2 @ rs.reshape(NCLASS, NCLASS))[None, :] + bd[None, :]
    return out


# speedup vs baseline: 1.2955x; 1.0089x over previous
import jax
import jax.numpy as jnp
from jax import lax
from jax.experimental import pallas as pl
from jax.experimental.pallas import tpu as pltpu

B, N, NFEAT, NHID, NCLASS = 4, 4096, 512, 512, 10
LANES = 128
RT = 512
NT = N // RT


def _gcn_kernel(x_ref, adj_ref, w1_ref, b1_ref, w2_ref, wdt_ref,
                p_ref, s1_ref, s2_ref, gt_ref):
    t = pl.program_id(1)

    @pl.when(t == 0)
    def _init():
        s1 = jnp.dot(x_ref[0].astype(jnp.bfloat16), w1_ref[...],
                     preferred_element_type=jnp.float32)
        s1_ref[...] = s1.astype(jnp.bfloat16)
        gt_ref[...] = jnp.zeros_like(gt_ref)

    adj_t = adj_ref[0].astype(jnp.bfloat16)

    z = jnp.dot(adj_t, s1_ref[...], preferred_element_type=jnp.float32)
    h1_t = jnp.maximum(z + b1_ref[...], 0.0).astype(jnp.bfloat16)

    s2_t = jnp.dot(h1_t, w2_ref[...], preferred_element_type=jnp.float32)
    s2_ref[pl.ds(t * RT, RT), :] = s2_t.astype(jnp.bfloat16)

    wdf_t = wdt_ref[pl.ds(t * RT, RT), :]
    gt_ref[...] += lax.dot_general(
        wdf_t, adj_t, (((0,), (0,)), ((), ())),
        preferred_element_type=jnp.float32)

    @pl.when(t == NT - 1)
    def _fin():
        p_ref[0] = jnp.dot(gt_ref[...].astype(jnp.bfloat16), s2_ref[...],
                           preferred_element_type=jnp.float32)


def kernel(x, adj, W1, b1, W2, b2, Wd, bd):
    w1 = W1.astype(jnp.bfloat16)
    w2 = jnp.pad(W2, ((0, 0), (0, LANES - NCLASS))).astype(jnp.bfloat16)
    wdt = jnp.pad(Wd.reshape(N, NCLASS * NCLASS),
                  ((0, 0), (0, LANES - NCLASS * NCLASS))).astype(jnp.bfloat16)
    b1r = b1.reshape(1, NHID)

    p = pl.pallas_call(
        _gcn_kernel,
        grid=(B, NT),
        in_specs=[
            pl.BlockSpec((1, N, NFEAT), lambda b, t: (b, 0, 0)),
            pl.BlockSpec((1, RT, N), lambda b, t: (b, t, 0)),
            pl.BlockSpec((NFEAT, NHID), lambda b, t: (0, 0)),
            pl.BlockSpec((1, NHID), lambda b, t: (0, 0)),
            pl.BlockSpec((NHID, LANES), lambda b, t: (0, 0)),
            pl.BlockSpec((N, LANES), lambda b, t: (0, 0)),
        ],
        out_specs=pl.BlockSpec((1, LANES, LANES), lambda b, t: (b, 0, 0)),
        out_shape=jax.ShapeDtypeStruct((B, LANES, LANES), jnp.float32),
        scratch_shapes=[
            pltpu.VMEM((N, NHID), jnp.bfloat16),
            pltpu.VMEM((N, LANES), jnp.bfloat16),
            pltpu.VMEM((LANES, N), jnp.float32),
        ],
        compiler_params=pltpu.CompilerParams(
            dimension_semantics=("parallel", "arbitrary"),
        ),
    )(x, adj, w1, b1r, w2, wdt)

    pr = p[:, :NCLASS * NCLASS, :NCLASS].reshape(B, NCLASS, NCLASS, NCLASS)
    out = jnp.einsum('bkck->bc', pr)
    rs = jnp.sum(wdt.astype(jnp.float32), axis=0)[:NCLASS * NCLASS]
    out = out + (b2 @ rs.reshape(NCLASS, NCLASS))[None, :] + bd[None, :]
    return out
